# Initial kernel scaffold; baseline (speedup 1.0000x reference)
import jax
import jax.numpy as jnp
from jax.experimental import pallas as pl


def kernel(x, pos, edge_index, batch, params):
    raise NotImplementedError("write your pallas kernel here")


# speedup vs baseline: 4.0141x; 4.0141x over previous
import functools

import jax
import jax.numpy as jnp
from jax import lax
from jax.experimental import pallas as pl
from jax.experimental.pallas import tpu as pltpu
from jax.experimental.pallas import tpu_sc as plsc

N = 100000
E = 1600000
G = 512
H = 128
F = 16
BN_EPS = 1e-5

NW = 8
NP = 100352
W = NP // NW
NWORK = 32
SHARD = E // NWORK
ECHUNK = 10000
NVREG = ECHUNK // 16
STAGE = 1088
FLUSH = 1024
CAP = 51200
ROWS_T = W // 16
C = 128
TRASH = W


def _worker_id():
    c = lax.axis_index("c")
    s = lax.axis_index("s")
    return s * 2 + c, c, s


def _bin_body(src_hbm, dst_hbm, sbin, dbin, cnt, ebuf_s, ebuf_d, sstage,
              dstage, cvec):
    w, c, s = _worker_id()
    lane = lax.iota(jnp.int32, 16)
    wv16 = jnp.broadcast_to(jnp.int32(w * 97), (16,))
    trash_src = (lane * 611 + wv16) % N
    trash_dst = jnp.full((16,), TRASH, jnp.int32)

    def vbody(j, carry):
        offs = carry[0:NW]
        fls = carry[NW:2 * NW]
        dstv = ebuf_d[pl.ds(j * 16, 16)]
        srcv = ebuf_s[pl.ds(j * 16, 16)]
        wv = dstv // W
        dloc = dstv - wv * W
        new_offs = []
        new_fls = []
        for p in range(NW):
            m = wv == p
            cntp = jnp.sum(m.astype(jnp.int32))
            plsc.store_compressed(sstage.at[pl.ds(p * STAGE + offs[p], 16)], srcv,
                                  mask=m)
            plsc.store_compressed(dstage.at[pl.ds(p * STAGE + offs[p], 16)], dloc,
                                  mask=m)
            off = offs[p] + cntp
            do_flush = off >= FLUSH

            @pl.when(do_flush)
            def _():
                bo = pl.multiple_of((p * NWORK + w) * CAP + fls[p], 8)
                pltpu.sync_copy(sstage.at[pl.ds(p * STAGE, FLUSH)],
                                sbin.at[pl.ds(bo, FLUSH)])
                pltpu.sync_copy(dstage.at[pl.ds(p * STAGE, FLUSH)],
                                dbin.at[pl.ds(bo, FLUSH)])
                sv = sstage[pl.ds(p * STAGE + FLUSH, 16)]
                dv = dstage[pl.ds(p * STAGE + FLUSH, 16)]
                sstage[pl.ds(p * STAGE, 16)] = sv
                dstage[pl.ds(p * STAGE, 16)] = dv

            new_offs.append(jnp.where(do_flush, off - FLUSH, off))
            new_fls.append(jnp.where(do_flush, fls[p] + FLUSH, fls[p]))
        return tuple(new_offs) + tuple(new_fls)

    carry = (jnp.int32(0),) * (2 * NW)
    for ch in range(SHARD // ECHUNK):
        ebase = pl.multiple_of(w * SHARD + ch * ECHUNK, 8)
        pltpu.sync_copy(src_hbm.at[pl.ds(ebase, ECHUNK)], ebuf_s)
        pltpu.sync_copy(dst_hbm.at[pl.ds(ebase, ECHUNK)], ebuf_d)
        carry = lax.fori_loop(0, NVREG, vbody, carry)

    offs = carry[0:NW]
    fls = carry[NW:2 * NW]
    counts16 = jnp.zeros((16,), jnp.int32)
    for p in range(NW):
        def padbody(k, _, p=p, off=offs[p]):
            rel = jnp.broadcast_to(off - k * 16, (16,))
            cs = sstage[pl.ds(p * STAGE + k * 16, 16)]
            cd = dstage[pl.ds(p * STAGE + k * 16, 16)]
            keep = lane < rel
            sstage[pl.ds(p * STAGE + k * 16, 16)] = jnp.where(
                keep, cs, trash_src)
            dstage[pl.ds(p * STAGE + k * 16, 16)] = jnp.where(
                keep, cd, trash_dst)
            return 0

        lax.fori_loop(0, FLUSH // 16 + 1, padbody, 0)
        bo = pl.multiple_of((p * NWORK + w) * CAP + fls[p], 8)
        pltpu.sync_copy(sstage.at[pl.ds(p * STAGE, FLUSH)],
                        sbin.at[pl.ds(bo, FLUSH)])
        pltpu.sync_copy(dstage.at[pl.ds(p * STAGE, FLUSH)],
                        dbin.at[pl.ds(bo, FLUSH)])
        counts16 = jnp.where(lane == p,
                             jnp.broadcast_to(fls[p] + FLUSH, (16,)),
                             counts16)
    cvec[...] = counts16
    pltpu.sync_copy(cvec, cnt.at[pl.ds(pl.multiple_of(w * 16, 8), 16)])


def _bin_edges(src, dst):
    mesh = plsc.VectorSubcoreMesh(core_axis_name="c", subcore_axis_name="s")
    fn = pl.kernel(
        _bin_body,
        out_type=(
            jax.ShapeDtypeStruct((NW * NWORK * CAP,), jnp.int32),
            jax.ShapeDtypeStruct((NW * NWORK * CAP,), jnp.int32),
            jax.ShapeDtypeStruct((NWORK * 16,), jnp.int32),
        ),
        mesh=mesh,
        compiler_params=pltpu.CompilerParams(needs_layout_passes=False),
        scratch_types=[
            pltpu.VMEM((ECHUNK,), jnp.int32),
            pltpu.VMEM((ECHUNK,), jnp.int32),
            pltpu.VMEM((NW * STAGE,), jnp.int32),
            pltpu.VMEM((NW * STAGE,), jnp.int32),
            pltpu.VMEM((16,), jnp.int32),
        ],
    )
    return fn(src, dst)


def _agg_body(hd, h_hbm, sbin, dbin, cnt, m_hbm, cvm, sidx, didx, gbuf, acc,
              sem):
    c = lax.axis_index("c")
    s = lax.axis_index("s")
    lane = lax.iota(jnp.int32, 16)
    pltpu.sync_copy(cnt, cvm)

    for p_local in range(NW // 2):
        p = c * (NW // 2) + p_local
        base = p * W
        row0 = pl.multiple_of(base + s * ROWS_T, 8)
        arow0 = pl.multiple_of(s * ROWS_T, 8)
        pltpu.sync_copy(h_hbm.at[pl.ds(row0, ROWS_T)],
                        acc.at[pl.ds(arow0, ROWS_T)])

        plsc.subcore_barrier()

        for bi in range(2):
            b = s * 2 + bi

            def chunk(j, _, p=p, b=b):
                bo = pl.multiple_of((p * NWORK + b) * CAP + j * C, 8)
                pltpu.sync_copy(sbin.at[pl.ds(bo, C)], sidx)
                pltpu.sync_copy(dbin.at[pl.ds(bo, C)], didx)
                pltpu.async_copy(h_hbm.at[sidx], gbuf, sem).wait()
                pltpu.sync_copy(gbuf, acc.at[didx], add=True)
                return 0

            cv = cvm[pl.ds(pl.multiple_of(b * 16, 8), 16)]
            pv = jnp.broadcast_to(p, (16,))
            n = jnp.sum(jnp.where(lane == pv, cv, 0))
            lax.fori_loop(0, n // C, chunk, 0)

        plsc.subcore_barrier()
        pltpu.sync_copy(acc.at[pl.ds(arow0, ROWS_T)],
                        m_hbm.at[pl.ds(row0, ROWS_T)])
        plsc.subcore_barrier()


def _aggregate(h, sbin, dbin, cnt, hd):
    mesh = plsc.VectorSubcoreMesh(core_axis_name="c", subcore_axis_name="s")
    fn = pl.kernel(
        functools.partial(_agg_body, hd),
        out_type=jax.ShapeDtypeStruct((NP, hd), jnp.float32),
        mesh=mesh,
        compiler_params=pltpu.CompilerParams(needs_layout_passes=False),
        scratch_types=[
            pltpu.VMEM((NWORK * 16,), jnp.int32),
            pltpu.VMEM((C,), jnp.int32),
            pltpu.VMEM((C,), jnp.int32),
            pltpu.VMEM((C, hd), jnp.float32),
            pltpu.VMEM_SHARED((W + 8, hd), jnp.float32),
            pltpu.SemaphoreType.DMA,
        ],
    )
    return fn(h, sbin, dbin, cnt)


BLK = 1024
NBLK = NP // BLK


def _mlp_body(m_ref, wa_ref, ba_ref, wb_ref, bb_ref, y_ref, st_ref):
    i = pl.program_id(0)
    t = jnp.maximum(
        jnp.dot(m_ref[...], wa_ref[...],
                preferred_element_type=jnp.float32) + ba_ref[...], 0.0)
    y = jnp.dot(t, wb_ref[...], preferred_element_type=jnp.float32) \
        + bb_ref[...]
    y_ref[...] = y

    @pl.when(i == 0)
    def _():
        st_ref[...] = jnp.zeros_like(st_ref)

    valid = (lax.broadcasted_iota(jnp.int32, (BLK, 1), 0)
             < N - i * BLK).astype(jnp.float32)
    yv = y * valid
    su = jnp.sum(yv, axis=0, keepdims=True)
    sq = jnp.sum(yv * yv, axis=0, keepdims=True)
    st_ref[...] = st_ref[...] + jnp.concatenate([su, sq], axis=0)


def _mlp(m, wa, ba, wb, bb):
    fd = m.shape[1]
    return pl.pallas_call(
        _mlp_body,
        grid=(NBLK,),
        in_specs=[
            pl.BlockSpec((BLK, fd), lambda i: (i, 0)),
            pl.BlockSpec((fd, H), lambda i: (0, 0)),
            pl.BlockSpec((1, H), lambda i: (0, 0)),
            pl.BlockSpec((H, H), lambda i: (0, 0)),
            pl.BlockSpec((1, H), lambda i: (0, 0)),
        ],
        out_specs=[
            pl.BlockSpec((BLK, H), lambda i: (i, 0)),
            pl.BlockSpec((2, H), lambda i: (0, 0)),
        ],
        out_shape=[
            jax.ShapeDtypeStruct((NP, H), jnp.float32),
            jax.ShapeDtypeStruct((2, H), jnp.float32),
        ],
    )(m, wa, ba, wb, bb)


def _premul_body(h_ref, wa_ref, z_ref):
    z_ref[...] = jnp.dot(h_ref[...], wa_ref[...],
                         preferred_element_type=jnp.float32)


def _premul(h0, wa):
    return pl.pallas_call(
        _premul_body,
        grid=(NBLK,),
        in_specs=[
            pl.BlockSpec((BLK, F), lambda i: (i, 0)),
            pl.BlockSpec((F, H), lambda i: (0, 0)),
        ],
        out_specs=pl.BlockSpec((BLK, H), lambda i: (i, 0)),
        out_shape=jax.ShapeDtypeStruct((NP, H), jnp.float32),
    )(h0, wa)


def _mlp1_body(m_ref, ba_ref, wb_ref, bb_ref, y_ref, st_ref):
    i = pl.program_id(0)
    t = jnp.maximum(m_ref[...] + ba_ref[...], 0.0)
    y = jnp.dot(t, wb_ref[...], preferred_element_type=jnp.float32) \
        + bb_ref[...]
    y_ref[...] = y

    @pl.when(i == 0)
    def _():
        st_ref[...] = jnp.zeros_like(st_ref)

    valid = (lax.broadcasted_iota(jnp.int32, (BLK, 1), 0)
             < N - i * BLK).astype(jnp.float32)
    yv = y * valid
    su = jnp.sum(yv, axis=0, keepdims=True)
    sq = jnp.sum(yv * yv, axis=0, keepdims=True)
    st_ref[...] = st_ref[...] + jnp.concatenate([su, sq], axis=0)


def _mlp1(m, ba, wb, bb):
    return pl.pallas_call(
        _mlp1_body,
        grid=(NBLK,),
        in_specs=[
            pl.BlockSpec((BLK, H), lambda i: (i, 0)),
            pl.BlockSpec((1, H), lambda i: (0, 0)),
            pl.BlockSpec((H, H), lambda i: (0, 0)),
            pl.BlockSpec((1, H), lambda i: (0, 0)),
        ],
        out_specs=[
            pl.BlockSpec((BLK, H), lambda i: (i, 0)),
            pl.BlockSpec((2, H), lambda i: (0, 0)),
        ],
        out_shape=[
            jax.ShapeDtypeStruct((NP, H), jnp.float32),
            jax.ShapeDtypeStruct((2, H), jnp.float32),
        ],
    )(m, ba, wb, bb)


def _bnrelu_body(y_ref, st_ref, g_ref, be_ref, h_ref):
    mean = st_ref[0:1, :] * (1.0 / N)
    ex2 = st_ref[1:2, :] * (1.0 / N)
    var = ex2 - mean * mean
    rstd = lax.rsqrt(var + BN_EPS)
    h_ref[...] = jnp.maximum(
        (y_ref[...] - mean) * rstd * g_ref[...] + be_ref[...], 0.0)


def _bnrelu(y, st, g, be):
    return pl.pallas_call(
        _bnrelu_body,
        grid=(NBLK,),
        in_specs=[
            pl.BlockSpec((BLK, H), lambda i: (i, 0)),
            pl.BlockSpec((2, H), lambda i: (0, 0)),
            pl.BlockSpec((1, H), lambda i: (0, 0)),
            pl.BlockSpec((1, H), lambda i: (0, 0)),
        ],
        out_specs=pl.BlockSpec((BLK, H), lambda i: (i, 0)),
        out_shape=jax.ShapeDtypeStruct((NP, H), jnp.float32),
    )(y, st, g, be)


def _pool_body(h_ref, bt_ref, fw1_ref, fb1_ref, fw2_ref, fb2_ref, out_ref,
               pool_acc, cnt_acc):
    i = pl.program_id(0)

    @pl.when(i == 0)
    def _():
        pool_acc[...] = jnp.zeros_like(pool_acc)
        cnt_acc[...] = jnp.zeros_like(cnt_acc)

    b = bt_ref[...].reshape(1, BLK)
    oh = (lax.broadcasted_iota(jnp.int32, (G, BLK), 0) == b).astype(
        jnp.float32)
    pool_acc[...] = pool_acc[...] + jnp.dot(
        oh, h_ref[...], preferred_element_type=jnp.float32)
    cnt_acc[...] = cnt_acc[...] + jnp.sum(oh, axis=1, keepdims=True)

    @pl.when(i == NBLK - 1)
    def _():
        mean = pool_acc[...] / jnp.maximum(cnt_acc[...], 1.0)
        o1 = jnp.maximum(
            jnp.dot(mean, fw1_ref[...],
                    preferred_element_type=jnp.float32) + fb1_ref[...], 0.0)
        out_ref[...] = jnp.dot(
            o1, fw2_ref[...], preferred_element_type=jnp.float32) \
            + fb2_ref[...]


def _pool_head(h, batch2d, fw1, fb1, fw2, fb2):
    return pl.pallas_call(
        _pool_body,
        grid=(NBLK,),
        in_specs=[
            pl.BlockSpec((BLK, H), lambda i: (i, 0)),
            pl.BlockSpec((1, 1, BLK), lambda i: (i, 0, 0)),
            pl.BlockSpec((H, H), lambda i: (0, 0)),
            pl.BlockSpec((1, H), lambda i: (0, 0)),
            pl.BlockSpec((H, 1), lambda i: (0, 0)),
            pl.BlockSpec((1, 1), lambda i: (0, 0)),
        ],
        out_specs=pl.BlockSpec((G, 1), lambda i: (0, 0)),
        out_shape=jax.ShapeDtypeStruct((G, 1), jnp.float32),
        scratch_shapes=[
            pltpu.VMEM((G, H), jnp.float32),
            pltpu.VMEM((G, 1), jnp.float32),
        ],
    )(h, batch2d, fw1, fb1, fw2, fb2)


def kernel(x, pos, edge_index, batch, params):
    p = params
    h0 = jnp.concatenate(
        [x, pos, jnp.zeros((N, F - 14), jnp.float32)], axis=1)
    h0 = jnp.concatenate(
        [h0, jnp.zeros((NP - N, F), jnp.float32)], axis=0)
    src = edge_index[0]
    dst = edge_index[1]
    sbin, dbin, cnt = _bin_edges(src, dst)

    w1a = jnp.concatenate(
        [p["w1a"], jnp.zeros((F - 14, H), jnp.float32)], axis=0)

    z0 = _premul(h0, w1a)
    m1 = _aggregate(z0, sbin, dbin, cnt, H)
    y1, st1 = _mlp1(m1, p["b1a"].reshape(1, H), p["w1b"],
                    p["b1b"].reshape(1, H))
    h1 = _bnrelu(y1, st1, p["g1"].reshape(1, H), p["be1"].reshape(1, H))

    m2 = _aggregate(h1, sbin, dbin, cnt, H)
    y2, st2 = _mlp(m2, p["w2a"], p["b2a"].reshape(1, H), p["w2b"],
                   p["b2b"].reshape(1, H))
    h2 = _bnrelu(y2, st2, p["g2"].reshape(1, H), p["be2"].reshape(1, H))

    m3 = _aggregate(h2, sbin, dbin, cnt, H)
    y3, st3 = _mlp(m3, p["w3a"], p["b3a"].reshape(1, H), p["w3b"],
                   p["b3b"].reshape(1, H))
    h3 = _bnrelu(y3, st3, p["g3"].reshape(1, H), p["be3"].reshape(1, H))

    batch_pad = jnp.concatenate(
        [batch.astype(jnp.int32), jnp.full((NP - N,), G, jnp.int32)])
    out = _pool_head(h3, batch_pad.reshape(NBLK, 1, BLK), p["fw1"],
                     p["fb1"].reshape(1, H), p["fw2"],
                     p["fb2"].reshape(1, 1))
    return out


# speedup vs baseline: 4.8555x; 1.2096x over previous
import functools

import jax
import jax.numpy as jnp
from jax import lax
from jax.experimental import pallas as pl
from jax.experimental.pallas import tpu as pltpu
from jax.experimental.pallas import tpu_sc as plsc

N = 100000
E = 1600000
G = 512
H = 128
F = 16
BN_EPS = 1e-5

NW = 14
NP = 100352
W = NP // NW
NWORK = 32
SHARD = E // NWORK
ECHUNK = 10000
NVREG = ECHUNK // 16
STAGE = 1088
FLUSH = 1024
CAP = 51200
ROWS_T = W // 16
C = 512
TRASH = W


def _worker_id():
    c = lax.axis_index("c")
    s = lax.axis_index("s")
    return s * 2 + c, c, s


def _bin_body(src_hbm, dst_hbm, sbin, dbin, cnt, ebuf_s, ebuf_d, sstage,
              dstage, cvec):
    w, c, s = _worker_id()
    lane = lax.iota(jnp.int32, 16)
    wv16 = jnp.broadcast_to(jnp.int32(w * 97), (16,))
    trash_src = (lane * 611 + wv16) % N
    trash_dst = jnp.full((16,), TRASH, jnp.int32)

    def vbody(j, carry):
        offs = carry[0:NW]
        fls = carry[NW:2 * NW]
        dstv = ebuf_d[pl.ds(j * 16, 16)]
        srcv = ebuf_s[pl.ds(j * 16, 16)]
        wv = dstv // W
        dloc = dstv - wv * W
        new_offs = []
        new_fls = []
        for p in range(NW):
            m = wv == p
            cntp = jnp.sum(m.astype(jnp.int32))
            plsc.store_compressed(sstage.at[pl.ds(p * STAGE + offs[p], 16)], srcv,
                                  mask=m)
            plsc.store_compressed(dstage.at[pl.ds(p * STAGE + offs[p], 16)], dloc,
                                  mask=m)
            off = offs[p] + cntp
            do_flush = off >= FLUSH

            @pl.when(do_flush)
            def _():
                bo = pl.multiple_of((p * NWORK + w) * CAP + fls[p], 8)
                pltpu.sync_copy(sstage.at[pl.ds(p * STAGE, FLUSH)],
                                sbin.at[pl.ds(bo, FLUSH)])
                pltpu.sync_copy(dstage.at[pl.ds(p * STAGE, FLUSH)],
                                dbin.at[pl.ds(bo, FLUSH)])
                sv = sstage[pl.ds(p * STAGE + FLUSH, 16)]
                dv = dstage[pl.ds(p * STAGE + FLUSH, 16)]
                sstage[pl.ds(p * STAGE, 16)] = sv
                dstage[pl.ds(p * STAGE, 16)] = dv

            new_offs.append(jnp.where(do_flush, off - FLUSH, off))
            new_fls.append(jnp.where(do_flush, fls[p] + FLUSH, fls[p]))
        return tuple(new_offs) + tuple(new_fls)

    carry = (jnp.int32(0),) * (2 * NW)
    for ch in range(SHARD // ECHUNK):
        ebase = pl.multiple_of(w * SHARD + ch * ECHUNK, 8)
        pltpu.sync_copy(src_hbm.at[pl.ds(ebase, ECHUNK)], ebuf_s)
        pltpu.sync_copy(dst_hbm.at[pl.ds(ebase, ECHUNK)], ebuf_d)
        carry = lax.fori_loop(0, NVREG, vbody, carry)

    offs = carry[0:NW]
    fls = carry[NW:2 * NW]
    counts16 = jnp.zeros((16,), jnp.int32)
    for p in range(NW):
        def padbody(k, _, p=p, off=offs[p]):
            rel = jnp.broadcast_to(off - k * 16, (16,))
            cs = sstage[pl.ds(p * STAGE + k * 16, 16)]
            cd = dstage[pl.ds(p * STAGE + k * 16, 16)]
            keep = lane < rel
            sstage[pl.ds(p * STAGE + k * 16, 16)] = jnp.where(
                keep, cs, trash_src)
            dstage[pl.ds(p * STAGE + k * 16, 16)] = jnp.where(
                keep, cd, trash_dst)
            return 0

        lax.fori_loop(0, FLUSH // 16 + 1, padbody, 0)
        bo = pl.multiple_of((p * NWORK + w) * CAP + fls[p], 8)
        pltpu.sync_copy(sstage.at[pl.ds(p * STAGE, FLUSH)],
                        sbin.at[pl.ds(bo, FLUSH)])
        pltpu.sync_copy(dstage.at[pl.ds(p * STAGE, FLUSH)],
                        dbin.at[pl.ds(bo, FLUSH)])
        counts16 = jnp.where(lane == p,
                             jnp.broadcast_to(fls[p] + FLUSH, (16,)),
                             counts16)
    cvec[...] = counts16
    pltpu.sync_copy(cvec, cnt.at[pl.ds(pl.multiple_of(w * 16, 8), 16)])


def _bin_edges(src, dst):
    mesh = plsc.VectorSubcoreMesh(core_axis_name="c", subcore_axis_name="s")
    fn = pl.kernel(
        _bin_body,
        out_type=(
            jax.ShapeDtypeStruct((NW * NWORK * CAP,), jnp.int32),
            jax.ShapeDtypeStruct((NW * NWORK * CAP,), jnp.int32),
            jax.ShapeDtypeStruct((NWORK * 16,), jnp.int32),
        ),
        mesh=mesh,
        compiler_params=pltpu.CompilerParams(needs_layout_passes=False),
        scratch_types=[
            pltpu.VMEM((ECHUNK,), jnp.int32),
            pltpu.VMEM((ECHUNK,), jnp.int32),
            pltpu.VMEM((NW * STAGE,), jnp.int32),
            pltpu.VMEM((NW * STAGE,), jnp.int32),
            pltpu.VMEM((16,), jnp.int32),
        ],
    )
    return fn(src, dst)


def _agg_body(hd, h_hbm, sbin, dbin, cnt, m_hbm, cvm, sidx, didx, gbuf, acc,
              sem):
    c = lax.axis_index("c")
    s = lax.axis_index("s")
    lane = lax.iota(jnp.int32, 16)
    pltpu.sync_copy(cnt, cvm)

    for p_local in range(NW // 2):
        p = c * (NW // 2) + p_local
        base = p * W
        row0 = pl.multiple_of(base + s * ROWS_T, 8)
        arow0 = pl.multiple_of(s * ROWS_T, 8)
        pltpu.sync_copy(h_hbm.at[pl.ds(row0, ROWS_T)],
                        acc.at[pl.ds(arow0, ROWS_T)])

        plsc.subcore_barrier()

        for bi in range(2):
            b = s * 2 + bi

            def chunk(j, _, p=p, b=b):
                bo = pl.multiple_of((p * NWORK + b) * CAP + j * C, 8)
                pltpu.sync_copy(sbin.at[pl.ds(bo, C)], sidx)
                pltpu.sync_copy(dbin.at[pl.ds(bo, C)], didx)
                pltpu.async_copy(h_hbm.at[sidx], gbuf, sem).wait()
                pltpu.sync_copy(gbuf, acc.at[didx], add=True)
                return 0

            cv = cvm[pl.ds(pl.multiple_of(b * 16, 8), 16)]
            pv = jnp.broadcast_to(p, (16,))
            n = jnp.sum(jnp.where(lane == pv, cv, 0))
            lax.fori_loop(0, n // C, chunk, 0)

        plsc.subcore_barrier()
        pltpu.sync_copy(acc.at[pl.ds(arow0, ROWS_T)],
                        m_hbm.at[pl.ds(row0, ROWS_T)])
        plsc.subcore_barrier()


def _aggregate(h, sbin, dbin, cnt, hd):
    mesh = plsc.VectorSubcoreMesh(core_axis_name="c", subcore_axis_name="s")
    fn = pl.kernel(
        functools.partial(_agg_body, hd),
        out_type=jax.ShapeDtypeStruct((NP, hd), jnp.float32),
        mesh=mesh,
        compiler_params=pltpu.CompilerParams(needs_layout_passes=False),
        scratch_types=[
            pltpu.VMEM((NWORK * 16,), jnp.int32),
            pltpu.VMEM((C,), jnp.int32),
            pltpu.VMEM((C,), jnp.int32),
            pltpu.VMEM((C, hd), jnp.float32),
            pltpu.VMEM_SHARED((W + 8, hd), jnp.float32),
            pltpu.SemaphoreType.DMA,
        ],
    )
    return fn(h, sbin, dbin, cnt)


BLK = 1024
NBLK = NP // BLK


def _mlp_body(m_ref, wa_ref, ba_ref, wb_ref, bb_ref, y_ref, st_ref):
    i = pl.program_id(0)
    t = jnp.maximum(
        jnp.dot(m_ref[...], wa_ref[...],
                preferred_element_type=jnp.float32) + ba_ref[...], 0.0)
    y = jnp.dot(t, wb_ref[...], preferred_element_type=jnp.float32) \
        + bb_ref[...]
    y_ref[...] = y

    @pl.when(i == 0)
    def _():
        st_ref[...] = jnp.zeros_like(st_ref)

    valid = (lax.broadcasted_iota(jnp.int32, (BLK, 1), 0)
             < N - i * BLK).astype(jnp.float32)
    yv = y * valid
    su = jnp.sum(yv, axis=0, keepdims=True)
    sq = jnp.sum(yv * yv, axis=0, keepdims=True)
    st_ref[...] = st_ref[...] + jnp.concatenate([su, sq], axis=0)


def _mlp(m, wa, ba, wb, bb):
    fd = m.shape[1]
    return pl.pallas_call(
        _mlp_body,
        grid=(NBLK,),
        in_specs=[
            pl.BlockSpec((BLK, fd), lambda i: (i, 0)),
            pl.BlockSpec((fd, H), lambda i: (0, 0)),
            pl.BlockSpec((1, H), lambda i: (0, 0)),
            pl.BlockSpec((H, H), lambda i: (0, 0)),
            pl.BlockSpec((1, H), lambda i: (0, 0)),
        ],
        out_specs=[
            pl.BlockSpec((BLK, H), lambda i: (i, 0)),
            pl.BlockSpec((2, H), lambda i: (0, 0)),
        ],
        out_shape=[
            jax.ShapeDtypeStruct((NP, H), jnp.float32),
            jax.ShapeDtypeStruct((2, H), jnp.float32),
        ],
    )(m, wa, ba, wb, bb)


def _premul_body(h_ref, wa_ref, z_ref):
    z_ref[...] = jnp.dot(h_ref[...], wa_ref[...],
                         preferred_element_type=jnp.float32)


def _premul(h0, wa):
    return pl.pallas_call(
        _premul_body,
        grid=(NBLK,),
        in_specs=[
            pl.BlockSpec((BLK, F), lambda i: (i, 0)),
            pl.BlockSpec((F, H), lambda i: (0, 0)),
        ],
        out_specs=pl.BlockSpec((BLK, H), lambda i: (i, 0)),
        out_shape=jax.ShapeDtypeStruct((NP, H), jnp.float32),
    )(h0, wa)


def _mlp1_body(m_ref, ba_ref, wb_ref, bb_ref, y_ref, st_ref):
    i = pl.program_id(0)
    t = jnp.maximum(m_ref[...] + ba_ref[...], 0.0)
    y = jnp.dot(t, wb_ref[...], preferred_element_type=jnp.float32) \
        + bb_ref[...]
    y_ref[...] = y

    @pl.when(i == 0)
    def _():
        st_ref[...] = jnp.zeros_like(st_ref)

    valid = (lax.broadcasted_iota(jnp.int32, (BLK, 1), 0)
             < N - i * BLK).astype(jnp.float32)
    yv = y * valid
    su = jnp.sum(yv, axis=0, keepdims=True)
    sq = jnp.sum(yv * yv, axis=0, keepdims=True)
    st_ref[...] = st_ref[...] + jnp.concatenate([su, sq], axis=0)


def _mlp1(m, ba, wb, bb):
    return pl.pallas_call(
        _mlp1_body,
        grid=(NBLK,),
        in_specs=[
            pl.BlockSpec((BLK, H), lambda i: (i, 0)),
            pl.BlockSpec((1, H), lambda i: (0, 0)),
            pl.BlockSpec((H, H), lambda i: (0, 0)),
            pl.BlockSpec((1, H), lambda i: (0, 0)),
        ],
        out_specs=[
            pl.BlockSpec((BLK, H), lambda i: (i, 0)),
            pl.BlockSpec((2, H), lambda i: (0, 0)),
        ],
        out_shape=[
            jax.ShapeDtypeStruct((NP, H), jnp.float32),
            jax.ShapeDtypeStruct((2, H), jnp.float32),
        ],
    )(m, ba, wb, bb)


def _bnrelu_body(y_ref, st_ref, g_ref, be_ref, h_ref):
    mean = st_ref[0:1, :] * (1.0 / N)
    ex2 = st_ref[1:2, :] * (1.0 / N)
    var = ex2 - mean * mean
    rstd = lax.rsqrt(var + BN_EPS)
    h_ref[...] = jnp.maximum(
        (y_ref[...] - mean) * rstd * g_ref[...] + be_ref[...], 0.0)


def _bnrelu(y, st, g, be):
    return pl.pallas_call(
        _bnrelu_body,
        grid=(NBLK,),
        in_specs=[
            pl.BlockSpec((BLK, H), lambda i: (i, 0)),
            pl.BlockSpec((2, H), lambda i: (0, 0)),
            pl.BlockSpec((1, H), lambda i: (0, 0)),
            pl.BlockSpec((1, H), lambda i: (0, 0)),
        ],
        out_specs=pl.BlockSpec((BLK, H), lambda i: (i, 0)),
        out_shape=jax.ShapeDtypeStruct((NP, H), jnp.float32),
    )(y, st, g, be)


def _pool_body(h_ref, bt_ref, fw1_ref, fb1_ref, fw2_ref, fb2_ref, out_ref,
               pool_acc, cnt_acc):
    i = pl.program_id(0)

    @pl.when(i == 0)
    def _():
        pool_acc[...] = jnp.zeros_like(pool_acc)
        cnt_acc[...] = jnp.zeros_like(cnt_acc)

    b = bt_ref[...].reshape(1, BLK)
    oh = (lax.broadcasted_iota(jnp.int32, (G, BLK), 0) == b).astype(
        jnp.float32)
    pool_acc[...] = pool_acc[...] + jnp.dot(
        oh, h_ref[...], preferred_element_type=jnp.float32)
    cnt_acc[...] = cnt_acc[...] + jnp.sum(oh, axis=1, keepdims=True)

    @pl.when(i == NBLK - 1)
    def _():
        mean = pool_acc[...] / jnp.maximum(cnt_acc[...], 1.0)
        o1 = jnp.maximum(
            jnp.dot(mean, fw1_ref[...],
                    preferred_element_type=jnp.float32) + fb1_ref[...], 0.0)
        out_ref[...] = jnp.dot(
            o1, fw2_ref[...], preferred_element_type=jnp.float32) \
            + fb2_ref[...]


def _pool_head(h, batch2d, fw1, fb1, fw2, fb2):
    return pl.pallas_call(
        _pool_body,
        grid=(NBLK,),
        in_specs=[
            pl.BlockSpec((BLK, H), lambda i: (i, 0)),
            pl.BlockSpec((1, 1, BLK), lambda i: (i, 0, 0)),
            pl.BlockSpec((H, H), lambda i: (0, 0)),
            pl.BlockSpec((1, H), lambda i: (0, 0)),
            pl.BlockSpec((H, 1), lambda i: (0, 0)),
            pl.BlockSpec((1, 1), lambda i: (0, 0)),
        ],
        out_specs=pl.BlockSpec((G, 1), lambda i: (0, 0)),
        out_shape=jax.ShapeDtypeStruct((G, 1), jnp.float32),
        scratch_shapes=[
            pltpu.VMEM((G, H), jnp.float32),
            pltpu.VMEM((G, 1), jnp.float32),
        ],
    )(h, batch2d, fw1, fb1, fw2, fb2)


def kernel(x, pos, edge_index, batch, params):
    p = params
    h0 = jnp.concatenate(
        [x, pos, jnp.zeros((N, F - 14), jnp.float32)], axis=1)
    h0 = jnp.concatenate(
        [h0, jnp.zeros((NP - N, F), jnp.float32)], axis=0)
    src = edge_index[0]
    dst = edge_index[1]
    sbin, dbin, cnt = _bin_edges(src, dst)

    w1a = jnp.concatenate(
        [p["w1a"], jnp.zeros((F - 14, H), jnp.float32)], axis=0)

    z0 = _premul(h0, w1a)
    m1 = _aggregate(z0, sbin, dbin, cnt, H)
    y1, st1 = _mlp1(m1, p["b1a"].reshape(1, H), p["w1b"],
                    p["b1b"].reshape(1, H))
    h1 = _bnrelu(y1, st1, p["g1"].reshape(1, H), p["be1"].reshape(1, H))

    m2 = _aggregate(h1, sbin, dbin, cnt, H)
    y2, st2 = _mlp(m2, p["w2a"], p["b2a"].reshape(1, H), p["w2b"],
                   p["b2b"].reshape(1, H))
    h2 = _bnrelu(y2, st2, p["g2"].reshape(1, H), p["be2"].reshape(1, H))

    m3 = _aggregate(h2, sbin, dbin, cnt, H)
    y3, st3 = _mlp(m3, p["w3a"], p["b3a"].reshape(1, H), p["w3b"],
                   p["b3b"].reshape(1, H))
    h3 = _bnrelu(y3, st3, p["g3"].reshape(1, H), p["be3"].reshape(1, H))

    batch_pad = jnp.concatenate(
        [batch.astype(jnp.int32), jnp.full((NP - N,), G, jnp.int32)])
    out = _pool_head(h3, batch_pad.reshape(NBLK, 1, BLK), p["fw1"],
                     p["fb1"].reshape(1, H), p["fw2"],
                     p["fb2"].reshape(1, 1))
    return out


# speedup vs baseline: 5.5445x; 1.1419x over previous
import functools

import jax
import jax.numpy as jnp
from jax import lax
from jax.experimental import pallas as pl
from jax.experimental.pallas import tpu as pltpu
from jax.experimental.pallas import tpu_sc as plsc

N = 100000
E = 1600000
G = 512
H = 128
F = 16
BN_EPS = 1e-5

NW = 14
NP = 100352
W = NP // NW
NWORK = 32
SHARD = E // NWORK
ECHUNK = 10000
NVREG = ECHUNK // 16
STAGE = 1088
FLUSH = 1024
CAP = 51200
ROWS_T = W // 16
C = 256
TRASH = W


def _worker_id():
    c = lax.axis_index("c")
    s = lax.axis_index("s")
    return s * 2 + c, c, s


def _bin_body(src_hbm, dst_hbm, sbin, dbin, cnt, ebuf_s, ebuf_d, sstage,
              dstage, cvec):
    w, c, s = _worker_id()
    lane = lax.iota(jnp.int32, 16)
    wv16 = jnp.broadcast_to(jnp.int32(w * 97), (16,))
    trash_src = (lane * 611 + wv16) % N
    trash_dst = jnp.full((16,), TRASH, jnp.int32)

    def vbody(j, carry):
        offs = carry[0:NW]
        fls = carry[NW:2 * NW]
        dstv = ebuf_d[pl.ds(j * 16, 16)]
        srcv = ebuf_s[pl.ds(j * 16, 16)]
        wv = dstv // W
        dloc = dstv - wv * W
        new_offs = []
        new_fls = []
        for p in range(NW):
            m = wv == p
            cntp = jnp.sum(m.astype(jnp.int32))
            plsc.store_compressed(sstage.at[pl.ds(p * STAGE + offs[p], 16)], srcv,
                                  mask=m)
            plsc.store_compressed(dstage.at[pl.ds(p * STAGE + offs[p], 16)], dloc,
                                  mask=m)
            off = offs[p] + cntp
            do_flush = off >= FLUSH

            @pl.when(do_flush)
            def _():
                bo = pl.multiple_of((p * NWORK + w) * CAP + fls[p], 8)
                pltpu.sync_copy(sstage.at[pl.ds(p * STAGE, FLUSH)],
                                sbin.at[pl.ds(bo, FLUSH)])
                pltpu.sync_copy(dstage.at[pl.ds(p * STAGE, FLUSH)],
                                dbin.at[pl.ds(bo, FLUSH)])
                sv = sstage[pl.ds(p * STAGE + FLUSH, 16)]
                dv = dstage[pl.ds(p * STAGE + FLUSH, 16)]
                sstage[pl.ds(p * STAGE, 16)] = sv
                dstage[pl.ds(p * STAGE, 16)] = dv

            new_offs.append(jnp.where(do_flush, off - FLUSH, off))
            new_fls.append(jnp.where(do_flush, fls[p] + FLUSH, fls[p]))
        return tuple(new_offs) + tuple(new_fls)

    carry = (jnp.int32(0),) * (2 * NW)
    for ch in range(SHARD // ECHUNK):
        ebase = pl.multiple_of(w * SHARD + ch * ECHUNK, 8)
        pltpu.sync_copy(src_hbm.at[pl.ds(ebase, ECHUNK)], ebuf_s)
        pltpu.sync_copy(dst_hbm.at[pl.ds(ebase, ECHUNK)], ebuf_d)
        carry = lax.fori_loop(0, NVREG, vbody, carry)

    offs = carry[0:NW]
    fls = carry[NW:2 * NW]
    counts16 = jnp.zeros((16,), jnp.int32)
    for p in range(NW):
        def padbody(k, _, p=p, off=offs[p]):
            rel = jnp.broadcast_to(off - k * 16, (16,))
            cs = sstage[pl.ds(p * STAGE + k * 16, 16)]
            cd = dstage[pl.ds(p * STAGE + k * 16, 16)]
            keep = lane < rel
            sstage[pl.ds(p * STAGE + k * 16, 16)] = jnp.where(
                keep, cs, trash_src)
            dstage[pl.ds(p * STAGE + k * 16, 16)] = jnp.where(
                keep, cd, trash_dst)
            return 0

        lax.fori_loop(0, FLUSH // 16 + 1, padbody, 0)
        bo = pl.multiple_of((p * NWORK + w) * CAP + fls[p], 8)
        pltpu.sync_copy(sstage.at[pl.ds(p * STAGE, FLUSH)],
                        sbin.at[pl.ds(bo, FLUSH)])
        pltpu.sync_copy(dstage.at[pl.ds(p * STAGE, FLUSH)],
                        dbin.at[pl.ds(bo, FLUSH)])
        counts16 = jnp.where(lane == p,
                             jnp.broadcast_to(fls[p] + FLUSH, (16,)),
                             counts16)
    cvec[...] = counts16
    pltpu.sync_copy(cvec, cnt.at[pl.ds(pl.multiple_of(w * 16, 8), 16)])


def _bin_edges(src, dst):
    mesh = plsc.VectorSubcoreMesh(core_axis_name="c", subcore_axis_name="s")
    fn = pl.kernel(
        _bin_body,
        out_type=(
            jax.ShapeDtypeStruct((NW * NWORK * CAP,), jnp.int32),
            jax.ShapeDtypeStruct((NW * NWORK * CAP,), jnp.int32),
            jax.ShapeDtypeStruct((NWORK * 16,), jnp.int32),
        ),
        mesh=mesh,
        compiler_params=pltpu.CompilerParams(needs_layout_passes=False),
        scratch_types=[
            pltpu.VMEM((ECHUNK,), jnp.int32),
            pltpu.VMEM((ECHUNK,), jnp.int32),
            pltpu.VMEM((NW * STAGE,), jnp.int32),
            pltpu.VMEM((NW * STAGE,), jnp.int32),
            pltpu.VMEM((16,), jnp.int32),
        ],
    )
    return fn(src, dst)


def _agg_body(hd, h_hbm, sbin, dbin, cnt, m_hbm, cvm, s0, d0, s1, d1,
              g0, g1, acc, sg0, sg1, ss0, ss1):
    c = lax.axis_index("c")
    s = lax.axis_index("s")
    lane = lax.iota(jnp.int32, 16)
    pltpu.sync_copy(cnt, cvm)
    sidx = (s0, s1)
    didx = (d0, d1)
    gbuf = (g0, g1)
    sg = (sg0, sg1)
    ss = (ss0, ss1)

    for p_local in range(NW // 2):
        p = c * (NW // 2) + p_local
        base = p * W
        row0 = pl.multiple_of(base + s * ROWS_T, 8)
        arow0 = pl.multiple_of(s * ROWS_T, 8)
        pltpu.sync_copy(h_hbm.at[pl.ds(row0, ROWS_T)],
                        acc.at[pl.ds(arow0, ROWS_T)])

        plsc.subcore_barrier()

        for bi in range(2):
            b = s * 2 + bi
            bo0 = (p * NWORK + b) * CAP
            cv = cvm[pl.ds(pl.multiple_of(b * 16, 8), 16)]
            pv = jnp.broadcast_to(p, (16,))
            n = jnp.sum(jnp.where(lane == pv, cv, 0))
            nch = n // C

            def pair(k, _, bo0=bo0, nch=nch):
                for t in (0, 1):
                    j = 2 * k + t
                    u = 1 - t

                    @pl.when(j < nch)
                    def _():
                        @pl.when(j >= 2)
                        def _():
                            pltpu.make_async_copy(
                                gbuf[t], acc.at[didx[t]], ss[t]).wait()
                        off = pl.multiple_of(bo0 + j * C, 8)
                        pltpu.sync_copy(sbin.at[pl.ds(off, C)], sidx[t])
                        pltpu.sync_copy(dbin.at[pl.ds(off, C)], didx[t])
                        pltpu.async_copy(h_hbm.at[sidx[t]], gbuf[t], sg[t])

                    @pl.when((j >= 1) & (j - 1 < nch))
                    def _():
                        pltpu.make_async_copy(
                            h_hbm.at[sidx[u]], gbuf[u], sg[u]).wait()
                        pltpu.async_copy(gbuf[u], acc.at[didx[u]], ss[u],
                                         add=True)
                return 0

            lax.fori_loop(0, nch // 2 + 1, pair, 0)
            pltpu.make_async_copy(gbuf[0], acc.at[didx[0]], ss[0]).wait()
            pltpu.make_async_copy(gbuf[1], acc.at[didx[1]], ss[1]).wait()

        plsc.subcore_barrier()
        pltpu.sync_copy(acc.at[pl.ds(arow0, ROWS_T)],
                        m_hbm.at[pl.ds(row0, ROWS_T)])
        plsc.subcore_barrier()


def _aggregate(h, sbin, dbin, cnt, hd):
    mesh = plsc.VectorSubcoreMesh(core_axis_name="c", subcore_axis_name="s")
    fn = pl.kernel(
        functools.partial(_agg_body, hd),
        out_type=jax.ShapeDtypeStruct((NP, hd), jnp.float32),
        mesh=mesh,
        compiler_params=pltpu.CompilerParams(needs_layout_passes=False),
        scratch_types=[
            pltpu.VMEM((NWORK * 16,), jnp.int32),
            pltpu.VMEM((C,), jnp.int32),
            pltpu.VMEM((C,), jnp.int32),
            pltpu.VMEM((C,), jnp.int32),
            pltpu.VMEM((C,), jnp.int32),
            pltpu.VMEM((C, hd), jnp.float32),
            pltpu.VMEM((C, hd), jnp.float32),
            pltpu.VMEM_SHARED((W + 8, hd), jnp.float32),
            pltpu.SemaphoreType.DMA,
            pltpu.SemaphoreType.DMA,
            pltpu.SemaphoreType.DMA,
            pltpu.SemaphoreType.DMA,
        ],
    )
    return fn(h, sbin, dbin, cnt)


BLK = 1024
NBLK = NP // BLK


def _mlp_body(m_ref, wa_ref, ba_ref, wb_ref, bb_ref, y_ref, st_ref):
    i = pl.program_id(0)
    t = jnp.maximum(
        jnp.dot(m_ref[...], wa_ref[...],
                preferred_element_type=jnp.float32) + ba_ref[...], 0.0)
    y = jnp.dot(t, wb_ref[...], preferred_element_type=jnp.float32) \
        + bb_ref[...]
    y_ref[...] = y

    @pl.when(i == 0)
    def _():
        st_ref[...] = jnp.zeros_like(st_ref)

    valid = (lax.broadcasted_iota(jnp.int32, (BLK, 1), 0)
             < N - i * BLK).astype(jnp.float32)
    yv = y * valid
    su = jnp.sum(yv, axis=0, keepdims=True)
    sq = jnp.sum(yv * yv, axis=0, keepdims=True)
    st_ref[...] = st_ref[...] + jnp.concatenate([su, sq], axis=0)


def _mlp(m, wa, ba, wb, bb):
    fd = m.shape[1]
    return pl.pallas_call(
        _mlp_body,
        grid=(NBLK,),
        in_specs=[
            pl.BlockSpec((BLK, fd), lambda i: (i, 0)),
            pl.BlockSpec((fd, H), lambda i: (0, 0)),
            pl.BlockSpec((1, H), lambda i: (0, 0)),
            pl.BlockSpec((H, H), lambda i: (0, 0)),
            pl.BlockSpec((1, H), lambda i: (0, 0)),
        ],
        out_specs=[
            pl.BlockSpec((BLK, H), lambda i: (i, 0)),
            pl.BlockSpec((2, H), lambda i: (0, 0)),
        ],
        out_shape=[
            jax.ShapeDtypeStruct((NP, H), jnp.float32),
            jax.ShapeDtypeStruct((2, H), jnp.float32),
        ],
    )(m, wa, ba, wb, bb)


def _premul_body(h_ref, wa_ref, z_ref):
    z_ref[...] = jnp.dot(h_ref[...], wa_ref[...],
                         preferred_element_type=jnp.float32)


def _premul(h0, wa):
    return pl.pallas_call(
        _premul_body,
        grid=(NBLK,),
        in_specs=[
            pl.BlockSpec((BLK, F), lambda i: (i, 0)),
            pl.BlockSpec((F, H), lambda i: (0, 0)),
        ],
        out_specs=pl.BlockSpec((BLK, H), lambda i: (i, 0)),
        out_shape=jax.ShapeDtypeStruct((NP, H), jnp.float32),
    )(h0, wa)


def _mlp1_body(m_ref, ba_ref, wb_ref, bb_ref, y_ref, st_ref):
    i = pl.program_id(0)
    t = jnp.maximum(m_ref[...] + ba_ref[...], 0.0)
    y = jnp.dot(t, wb_ref[...], preferred_element_type=jnp.float32) \
        + bb_ref[...]
    y_ref[...] = y

    @pl.when(i == 0)
    def _():
        st_ref[...] = jnp.zeros_like(st_ref)

    valid = (lax.broadcasted_iota(jnp.int32, (BLK, 1), 0)
             < N - i * BLK).astype(jnp.float32)
    yv = y * valid
    su = jnp.sum(yv, axis=0, keepdims=True)
    sq = jnp.sum(yv * yv, axis=0, keepdims=True)
    st_ref[...] = st_ref[...] + jnp.concatenate([su, sq], axis=0)


def _mlp1(m, ba, wb, bb):
    return pl.pallas_call(
        _mlp1_body,
        grid=(NBLK,),
        in_specs=[
            pl.BlockSpec((BLK, H), lambda i: (i, 0)),
            pl.BlockSpec((1, H), lambda i: (0, 0)),
            pl.BlockSpec((H, H), lambda i: (0, 0)),
            pl.BlockSpec((1, H), lambda i: (0, 0)),
        ],
        out_specs=[
            pl.BlockSpec((BLK, H), lambda i: (i, 0)),
            pl.BlockSpec((2, H), lambda i: (0, 0)),
        ],
        out_shape=[
            jax.ShapeDtypeStruct((NP, H), jnp.float32),
            jax.ShapeDtypeStruct((2, H), jnp.float32),
        ],
    )(m, ba, wb, bb)


def _bnrelu_body(y_ref, st_ref, g_ref, be_ref, h_ref):
    mean = st_ref[0:1, :] * (1.0 / N)
    ex2 = st_ref[1:2, :] * (1.0 / N)
    var = ex2 - mean * mean
    rstd = lax.rsqrt(var + BN_EPS)
    h_ref[...] = jnp.maximum(
        (y_ref[...] - mean) * rstd * g_ref[...] + be_ref[...], 0.0)


def _bnrelu(y, st, g, be):
    return pl.pallas_call(
        _bnrelu_body,
        grid=(NBLK,),
        in_specs=[
            pl.BlockSpec((BLK, H), lambda i: (i, 0)),
            pl.BlockSpec((2, H), lambda i: (0, 0)),
            pl.BlockSpec((1, H), lambda i: (0, 0)),
            pl.BlockSpec((1, H), lambda i: (0, 0)),
        ],
        out_specs=pl.BlockSpec((BLK, H), lambda i: (i, 0)),
        out_shape=jax.ShapeDtypeStruct((NP, H), jnp.float32),
    )(y, st, g, be)


def _pool_body(h_ref, bt_ref, fw1_ref, fb1_ref, fw2_ref, fb2_ref, out_ref,
               pool_acc, cnt_acc):
    i = pl.program_id(0)

    @pl.when(i == 0)
    def _():
        pool_acc[...] = jnp.zeros_like(pool_acc)
        cnt_acc[...] = jnp.zeros_like(cnt_acc)

    b = bt_ref[...].reshape(1, BLK)
    oh = (lax.broadcasted_iota(jnp.int32, (G, BLK), 0) == b).astype(
        jnp.float32)
    pool_acc[...] = pool_acc[...] + jnp.dot(
        oh, h_ref[...], preferred_element_type=jnp.float32)
    cnt_acc[...] = cnt_acc[...] + jnp.sum(oh, axis=1, keepdims=True)

    @pl.when(i == NBLK - 1)
    def _():
        mean = pool_acc[...] / jnp.maximum(cnt_acc[...], 1.0)
        o1 = jnp.maximum(
            jnp.dot(mean, fw1_ref[...],
                    preferred_element_type=jnp.float32) + fb1_ref[...], 0.0)
        out_ref[...] = jnp.dot(
            o1, fw2_ref[...], preferred_element_type=jnp.float32) \
            + fb2_ref[...]


def _pool_head(h, batch2d, fw1, fb1, fw2, fb2):
    return pl.pallas_call(
        _pool_body,
        grid=(NBLK,),
        in_specs=[
            pl.BlockSpec((BLK, H), lambda i: (i, 0)),
            pl.BlockSpec((1, 1, BLK), lambda i: (i, 0, 0)),
            pl.BlockSpec((H, H), lambda i: (0, 0)),
            pl.BlockSpec((1, H), lambda i: (0, 0)),
            pl.BlockSpec((H, 1), lambda i: (0, 0)),
            pl.BlockSpec((1, 1), lambda i: (0, 0)),
        ],
        out_specs=pl.BlockSpec((G, 1), lambda i: (0, 0)),
        out_shape=jax.ShapeDtypeStruct((G, 1), jnp.float32),
        scratch_shapes=[
            pltpu.VMEM((G, H), jnp.float32),
            pltpu.VMEM((G, 1), jnp.float32),
        ],
    )(h, batch2d, fw1, fb1, fw2, fb2)


def kernel(x, pos, edge_index, batch, params):
    p = params
    h0 = jnp.concatenate(
        [x, pos, jnp.zeros((N, F - 14), jnp.float32)], axis=1)
    h0 = jnp.concatenate(
        [h0, jnp.zeros((NP - N, F), jnp.float32)], axis=0)
    src = edge_index[0]
    dst = edge_index[1]
    sbin, dbin, cnt = _bin_edges(src, dst)

    w1a = jnp.concatenate(
        [p["w1a"], jnp.zeros((F - 14, H), jnp.float32)], axis=0)

    z0 = _premul(h0, w1a)
    m1 = _aggregate(z0, sbin, dbin, cnt, H)
    y1, st1 = _mlp1(m1, p["b1a"].reshape(1, H), p["w1b"],
                    p["b1b"].reshape(1, H))
    h1 = _bnrelu(y1, st1, p["g1"].reshape(1, H), p["be1"].reshape(1, H))

    m2 = _aggregate(h1, sbin, dbin, cnt, H)
    y2, st2 = _mlp(m2, p["w2a"], p["b2a"].reshape(1, H), p["w2b"],
                   p["b2b"].reshape(1, H))
    h2 = _bnrelu(y2, st2, p["g2"].reshape(1, H), p["be2"].reshape(1, H))

    m3 = _aggregate(h2, sbin, dbin, cnt, H)
    y3, st3 = _mlp(m3, p["w3a"], p["b3a"].reshape(1, H), p["w3b"],
                   p["b3b"].reshape(1, H))
    h3 = _bnrelu(y3, st3, p["g3"].reshape(1, H), p["be3"].reshape(1, H))

    batch_pad = jnp.concatenate(
        [batch.astype(jnp.int32), jnp.full((NP - N,), G, jnp.int32)])
    out = _pool_head(h3, batch_pad.reshape(NBLK, 1, BLK), p["fw1"],
                     p["fb1"].reshape(1, H), p["fw2"],
                     p["fb2"].reshape(1, 1))
    return out
